# baseline (device time: 28769 ns/iter reference)
import jax
import jax.numpy as jnp
from jax import lax
from jax.experimental import pallas as pl
from jax.experimental.pallas import tpu as pltpu

N_DEV = 4
N_HOPS = N_DEV - 1
SEED = N_HOPS
G = 8


def kernel(x):
    m_per, n = x.shape
    n_grp = m_per // G

    def body(x_ref, out_ref, comm_ref, send_sems, recv_sems):
        my_pos = lax.axis_index("i")
        left = lax.rem(my_pos + N_DEV - 1, N_DEV)
        right = lax.rem(my_pos + 1, N_DEV)

        barrier_sem = pltpu.get_barrier_semaphore()
        for nbr in (left, right):
            pl.semaphore_signal(
                barrier_sem, inc=1,
                device_id=(nbr,), device_id_type=pl.DeviceIdType.MESH,
            )
        pl.semaphore_wait(barrier_sem, 2)

        def hop(h):
            src = SEED if h == 0 else h - 1
            return pltpu.make_async_remote_copy(
                src_ref=comm_ref.at[src],
                dst_ref=comm_ref.at[h],
                send_sem=send_sems.at[h],
                recv_sem=recv_sems.at[h],
                device_id=(right,),
                device_id_type=pl.DeviceIdType.MESH,
            )

        t = x_ref[:, :]
        rows = m_per
        while rows > 1:
            half = rows // 2
            t = t[:half, :] * t[half:rows, :]
            rows = half
        comm_ref[SEED, :, :] = t
        r0 = hop(0)
        r0.start()

        y = x_ref[:, :].reshape(n_grp, G, n)
        shift = 1
        while shift < G:
            pad = jnp.ones((n_grp, shift, n), dtype=y.dtype)
            y = y * jnp.concatenate([pad, y[:, : G - shift, :]], axis=1)
            shift *= 2

        r0.wait_recv()
        r1 = hop(1)
        r1.start()

        e = y[:, G - 1, :]
        shift = 1
        while shift < n_grp:
            pad = jnp.ones((shift, n), dtype=e.dtype)
            e = e * jnp.concatenate([pad, e[: n_grp - shift, :]], axis=0)
            shift *= 2
        e_excl = jnp.concatenate(
            [jnp.ones((1, n), dtype=e.dtype), e[: n_grp - 1, :]], axis=0
        )

        r1.wait_recv()
        r2 = hop(2)
        r2.start()

        z = y * e_excl[:, None, :]

        r2.wait_recv()

        prefix = jnp.ones((1, n), dtype=x_ref.dtype)
        for h in range(N_HOPS):
            v = comm_ref[h, :, :]
            prefix = prefix * jnp.where(h < my_pos, v, jnp.ones_like(v))

        out_ref[:, :] = z.reshape(m_per, n) * prefix

        r0.wait_send()
        r1.wait_send()
        r2.wait_send()

    return pl.pallas_call(
        body,
        out_shape=jax.ShapeDtypeStruct((m_per, n), x.dtype),
        in_specs=[pl.BlockSpec(memory_space=pltpu.VMEM)],
        out_specs=pl.BlockSpec(memory_space=pltpu.VMEM),
        scratch_shapes=[
            pltpu.VMEM((N_HOPS + 1, 1, n), x.dtype),
            pltpu.SemaphoreType.DMA((N_HOPS,)),
            pltpu.SemaphoreType.DMA((N_HOPS,)),
        ],
        compiler_params=pltpu.CompilerParams(collective_id=0),
    )(x)


# device time: 28765 ns/iter; 1.0001x vs baseline; 1.0001x over previous
import jax
import jax.numpy as jnp
from jax import lax
from jax.experimental import pallas as pl
from jax.experimental.pallas import tpu as pltpu

N_DEV = 4
N_HOPS = N_DEV - 1
SEED = N_HOPS
G = 8


def kernel(x):
    m_per, n = x.shape
    n_grp = m_per // G

    def body(x_ref, out_ref, comm_ref, send_sems, recv_sems):
        my_pos = lax.axis_index("i")
        left = lax.rem(my_pos + N_DEV - 1, N_DEV)
        right = lax.rem(my_pos + 1, N_DEV)

        barrier_sem = pltpu.get_barrier_semaphore()
        for nbr in (left, right):
            pl.semaphore_signal(
                barrier_sem, inc=1,
                device_id=(nbr,), device_id_type=pl.DeviceIdType.MESH,
            )
        pl.semaphore_wait(barrier_sem, 2)

        def hop(h):
            src = SEED if h == 0 else h - 1
            return pltpu.make_async_remote_copy(
                src_ref=comm_ref.at[src],
                dst_ref=comm_ref.at[h],
                send_sem=send_sems.at[h],
                recv_sem=recv_sems.at[h],
                device_id=(right,),
                device_id_type=pl.DeviceIdType.MESH,
            )

        with jax.named_scope("seed"):
            t = x_ref[:, :]
            rows = m_per
            while rows > 1:
                half = rows // 2
                t = t[:half, :] * t[half:rows, :]
                rows = half
            comm_ref[SEED, :, :] = t
            r0 = hop(0)
            r0.start()

        with jax.named_scope("phaseA"):
            y = x_ref[:, :].reshape(n_grp, G, n)
            shift = 1
            while shift < G:
                pad = jnp.ones((n_grp, shift, n), dtype=y.dtype)
                y = y * jnp.concatenate([pad, y[:, : G - shift, :]], axis=1)
                shift *= 2

        with jax.named_scope("wait0"):
            r0.wait_recv()
            r1 = hop(1)
            r1.start()

        with jax.named_scope("phaseB"):
            e = y[:, G - 1, :]
            shift = 1
            while shift < n_grp:
                pad = jnp.ones((shift, n), dtype=e.dtype)
                e = e * jnp.concatenate([pad, e[: n_grp - shift, :]], axis=0)
                shift *= 2
            e_excl = jnp.concatenate(
                [jnp.ones((1, n), dtype=e.dtype), e[: n_grp - 1, :]], axis=0
            )

        with jax.named_scope("wait1"):
            r1.wait_recv()
            r2 = hop(2)
            r2.start()

        with jax.named_scope("phaseC"):
            z = y * e_excl[:, None, :]

        with jax.named_scope("wait2"):
            r2.wait_recv()

        with jax.named_scope("final"):
            prefix = jnp.ones((1, n), dtype=x_ref.dtype)
            for h in range(N_HOPS):
                v = comm_ref[h, :, :]
                prefix = prefix * jnp.where(h < my_pos, v, jnp.ones_like(v))

            out_ref[:, :] = z.reshape(m_per, n) * prefix

            r0.wait_send()
            r1.wait_send()
            r2.wait_send()

    return pl.pallas_call(
        body,
        out_shape=jax.ShapeDtypeStruct((m_per, n), x.dtype),
        in_specs=[pl.BlockSpec(memory_space=pltpu.VMEM)],
        out_specs=pl.BlockSpec(memory_space=pltpu.VMEM),
        scratch_shapes=[
            pltpu.VMEM((N_HOPS + 1, 1, n), x.dtype),
            pltpu.SemaphoreType.DMA((N_HOPS,)),
            pltpu.SemaphoreType.DMA((N_HOPS,)),
        ],
        compiler_params=pltpu.CompilerParams(collective_id=0),
    )(x)


# device time: 20342 ns/iter; 1.4143x vs baseline; 1.4141x over previous
import jax
import jax.numpy as jnp
from jax import lax
from jax.experimental import pallas as pl
from jax.experimental.pallas import tpu as pltpu

N_DEV = 4
N_HOPS = N_DEV - 1
SEED = N_HOPS
G = 8


def kernel(x):
    m_per, n = x.shape
    n_grp = m_per // G

    def body(x_ref, out_ref, comm_ref, send_sems, recv_sems):
        my_pos = lax.axis_index("i")
        left = lax.rem(my_pos + N_DEV - 1, N_DEV)
        right = lax.rem(my_pos + 1, N_DEV)

        barrier_sem = pltpu.get_barrier_semaphore()
        for nbr in (left, right):
            pl.semaphore_signal(
                barrier_sem, inc=1,
                device_id=(nbr,), device_id_type=pl.DeviceIdType.MESH,
            )
        pl.semaphore_wait(barrier_sem, 2)

        def hop(h):
            src = SEED if h == 0 else h - 1
            return pltpu.make_async_remote_copy(
                src_ref=comm_ref.at[src],
                dst_ref=comm_ref.at[h],
                send_sem=send_sems.at[h],
                recv_sem=recv_sems.at[h],
                device_id=(right,),
                device_id_type=pl.DeviceIdType.MESH,
            )

        with jax.named_scope("seed"):
            t = x_ref[:, :]
            rows = m_per
            while rows > 1:
                half = rows // 2
                t = t[:half, :] * t[half:rows, :]
                rows = half
            comm_ref[SEED, :, :] = t
            r0 = hop(0)
            r0.start()

        with jax.named_scope("phaseA"):
            y = x_ref[:, :]
            for shift in (1, 2, 4, 8, 16, 32):
                pad = jnp.ones((shift, n), dtype=y.dtype)
                y = y * jnp.concatenate([pad, y[: m_per - shift, :]], axis=0)

        with jax.named_scope("wait0"):
            r0.wait_recv()
            r1 = hop(1)
            r1.start()

        with jax.named_scope("phaseB"):
            for shift in (64, 128, 256):
                pad = jnp.ones((shift, n), dtype=y.dtype)
                y = y * jnp.concatenate([pad, y[: m_per - shift, :]], axis=0)

        with jax.named_scope("wait1"):
            r1.wait_recv()
            r2 = hop(2)
            r2.start()

        with jax.named_scope("phaseC"):
            for shift in (512, 1024):
                pad = jnp.ones((shift, n), dtype=y.dtype)
                y = y * jnp.concatenate([pad, y[: m_per - shift, :]], axis=0)
            z = y

        with jax.named_scope("wait2"):
            r2.wait_recv()

        with jax.named_scope("final"):
            prefix = jnp.ones((1, n), dtype=x_ref.dtype)
            for h in range(N_HOPS):
                v = comm_ref[h, :, :]
                prefix = prefix * jnp.where(h < my_pos, v, jnp.ones_like(v))

            out_ref[:, :] = z * prefix

            r0.wait_send()
            r1.wait_send()
            r2.wait_send()

    return pl.pallas_call(
        body,
        out_shape=jax.ShapeDtypeStruct((m_per, n), x.dtype),
        in_specs=[pl.BlockSpec(memory_space=pltpu.VMEM)],
        out_specs=pl.BlockSpec(memory_space=pltpu.VMEM),
        scratch_shapes=[
            pltpu.VMEM((N_HOPS + 1, 1, n), x.dtype),
            pltpu.SemaphoreType.DMA((N_HOPS,)),
            pltpu.SemaphoreType.DMA((N_HOPS,)),
        ],
        compiler_params=pltpu.CompilerParams(collective_id=0),
    )(x)
